# baseline (speedup 1.0000x reference)
import functools

import jax
import jax.numpy as jnp
from jax import lax
from jax.experimental import pallas as pl
from jax.experimental.pallas import tpu as pltpu
from jax.experimental.pallas import tpu_sc as plsc

CAP = 100000
DIM = 64
BATCH = 16384

NW = 32
SLAB = CAP // NW
LANES = 16
NG_SLAB = (SLAB + LANES - 1) // LANES
POS_PAD = NG_SLAB * LANES
NG_IDX = BATCH // LANES
CHUNK = 128
MAXCH = (SLAB + CHUNK - 1) // CHUNK
LIST_PAD = MAXCH * CHUNK
NG_LIST = LIST_PAD // LANES

_INT_MIN = -2147483647 - 1


def _sc_body(mem_hbm, idx_hbm, val_hbm, out_hbm,
             idx_v, pos_v, row1_v, win1_v, row2_v, win2_v, vbuf_v,
             copy_sem):
    wid = lax.axis_index("s") * 2 + lax.axis_index("c")
    base = wid * SLAB

    copy = pltpu.make_async_copy(
        mem_hbm.at[pl.ds(base, SLAB)], out_hbm.at[pl.ds(base, SLAB)], copy_sem)
    copy.start()

    with jax.named_scope("stage_idx"):
        pltpu.sync_copy(idx_hbm, idx_v)

    iota = lax.iota(jnp.int32, LANES)

    neg1 = jnp.full((LANES,), -1, jnp.int32)

    with jax.named_scope("init_pos"):
        @pl.loop(0, POS_PAD, step=LANES)
        def _(off):
            pos_v[pl.ds(off, LANES)] = neg1

    with jax.named_scope("scan"):
        @pl.loop(0, BATCH, step=LANES)
        def _(off):
            v = idx_v[pl.ds(off, LANES)]
            loc = v - base
            m = (loc >= 0) & (loc < SLAB)
            loc = jnp.where(m, loc, 0)
            plsc.store_scatter(pos_v, [loc], iota + off, mask=m)

    def _extract(g, cnt):
        p = pos_v[pl.ds(g * LANES, LANES)]
        m = p >= 0
        rows = iota + (base + g * LANES)
        plsc.store_compressed(row1_v.at[pl.ds(cnt, LANES)], rows, mask=m)
        plsc.store_compressed(win1_v.at[pl.ds(cnt, LANES)], p, mask=m)
        npop = jnp.max(plsc.all_reduce_population_count(m))
        return cnt + npop

    with jax.named_scope("extract"):
        cnt = lax.fori_loop(0, NG_SLAB, _extract, jnp.int32(0))
    nch = (cnt + CHUNK - 1) // CHUNK

    int_min = jnp.int32(_INT_MIN)
    r0 = jnp.max(jnp.where(iota == 0, row1_v[pl.ds(0, LANES)], int_min))
    w0 = jnp.max(jnp.where(iota == 0, win1_v[pl.ds(0, LANES)], int_min))

    with jax.named_scope("fill_pad"):
        @pl.loop(0, NG_LIST)
        def _(g):
            keep = (iota + g * LANES) < cnt
            rv = jnp.where(keep, row1_v[pl.ds(g * LANES, LANES)], r0)
            wv = jnp.where(keep, win1_v[pl.ds(g * LANES, LANES)], w0)
            j = g // (CHUNK // LANES)
            c = (g % (CHUNK // LANES)) * LANES
            row2_v[j, pl.ds(c, LANES)] = rv
            win2_v[j, pl.ds(c, LANES)] = wv

    with jax.named_scope("copy_wait"):
        copy.wait()

    with jax.named_scope("gs"):
        @pl.loop(0, MAXCH)
        def _(j):
            @pl.when(j < nch)
            def _():
                pltpu.sync_copy(val_hbm.at[win2_v.at[j]], vbuf_v)
                pltpu.sync_copy(vbuf_v, out_hbm.at[row2_v.at[j]])


@jax.jit
def _scatter_sc(mem, idx32, val):
    mesh = plsc.VectorSubcoreMesh(core_axis_name="c", subcore_axis_name="s")
    kfn = pl.kernel(
        _sc_body,
        out_type=jax.ShapeDtypeStruct((CAP, DIM), mem.dtype),
        mesh=mesh,
        compiler_params=pltpu.CompilerParams(
            use_tc_tiling_on_sc=False, needs_layout_passes=False),
        scratch_types=[
            pltpu.VMEM((BATCH,), jnp.int32),
            pltpu.VMEM((POS_PAD,), jnp.int32),
            pltpu.VMEM((LIST_PAD,), jnp.int32),
            pltpu.VMEM((LIST_PAD,), jnp.int32),
            pltpu.VMEM((MAXCH, CHUNK), jnp.int32),
            pltpu.VMEM((MAXCH, CHUNK), jnp.int32),
            pltpu.VMEM((CHUNK, DIM), jnp.float32),
            pltpu.SemaphoreType.DMA,
        ],
    )
    return kfn(mem, idx32, val)


def kernel(mem, idx, val):
    return _scatter_sc(mem, idx.astype(jnp.int32), val)


# speedup vs baseline: 4.7481x; 4.7481x over previous
import jax
import jax.numpy as jnp
from jax import lax
from jax.experimental import pallas as pl
from jax.experimental.pallas import tpu as pltpu
from jax.experimental.pallas import tpu_sc as plsc

CAP = 100000
DIM = 64
BATCH = 16384

NW = 32
SLAB = 3128
SLAB_LAST = CAP - (NW - 1) * SLAB
LANES = 16
POS_PAD = 3136
NG_SLAB = POS_PAD // LANES
CHUNK = 128
MAXCH = (SLAB + CHUNK - 1) // CHUNK
LIST_PAD = MAXCH * CHUNK
NG_LIST = LIST_PAD // LANES

CCH = 392
CTAIL = SLAB - 7 * CCH
CTAIL_LAST = SLAB_LAST - 7 * CCH

_INT_MIN = -2147483647 - 1


def _sc_body(mem_hbm, idx_hbm, val_hbm, out_hbm,
             idx_v, pos_v, row1_v, win1_v, row2_v, win2_v, vbuf_v, cbuf_v,
             isem0, isem1, osem0, osem1):
    wid = lax.axis_index("s") * 2 + lax.axis_index("c")
    base = wid * SLAB
    is_last = wid == NW - 1
    slab_len = jnp.where(is_last, SLAB_LAST, SLAB)

    isems = (isem0, isem1)
    osems = (osem0, osem1)

    def cin(c, n):
        b = c % 2
        return pltpu.make_async_copy(
            mem_hbm.at[pl.ds(base + c * CCH, n)],
            cbuf_v.at[b, pl.ds(0, n)], isems[b])

    def cout(c, n):
        b = c % 2
        return pltpu.make_async_copy(
            cbuf_v.at[b, pl.ds(0, n)],
            out_hbm.at[pl.ds(base + c * CCH, n)], osems[b])

    cin(0, CCH).start()
    cin(1, CCH).start()

    with jax.named_scope("stage_idx"):
        pltpu.sync_copy(idx_hbm, idx_v)

    iota = lax.iota(jnp.int32, LANES)

    neg1 = jnp.full((LANES,), -1, jnp.int32)

    with jax.named_scope("init_pos"):
        @pl.loop(0, POS_PAD, step=LANES)
        def _(off):
            pos_v[pl.ds(off, LANES)] = neg1

    with jax.named_scope("scan"):
        @pl.loop(0, BATCH, step=LANES)
        def _(off):
            v = idx_v[pl.ds(off, LANES)]
            loc = v - base
            m = (loc >= 0) & (loc < slab_len)
            loc = jnp.where(m, loc, 0)
            plsc.store_scatter(pos_v, [loc], iota + off, mask=m)

    def _extract(g, cnt):
        p = pos_v[pl.ds(g * LANES, LANES)]
        m = p >= 0
        rows = iota + (base + g * LANES)
        plsc.store_compressed(row1_v.at[pl.ds(cnt, LANES)], rows, mask=m)
        plsc.store_compressed(win1_v.at[pl.ds(cnt, LANES)], p, mask=m)
        npop = jnp.max(plsc.all_reduce_population_count(m))
        return cnt + npop

    with jax.named_scope("extract"):
        cnt = lax.fori_loop(0, NG_SLAB, _extract, jnp.int32(0))
    nch = (cnt + CHUNK - 1) // CHUNK

    int_min = jnp.int32(_INT_MIN)
    r0 = jnp.max(jnp.where(iota == 0, row1_v[pl.ds(0, LANES)], int_min))
    w0 = jnp.max(jnp.where(iota == 0, win1_v[pl.ds(0, LANES)], int_min))

    with jax.named_scope("fill_pad"):
        @pl.loop(0, NG_LIST)
        def _(g):
            keep = (iota + g * LANES) < cnt
            rv = jnp.where(keep, row1_v[pl.ds(g * LANES, LANES)], r0)
            wv = jnp.where(keep, win1_v[pl.ds(g * LANES, LANES)], w0)
            j = g // (CHUNK // LANES)
            c = (g % (CHUNK // LANES)) * LANES
            row2_v[j, pl.ds(c, LANES)] = rv
            win2_v[j, pl.ds(c, LANES)] = wv

    with jax.named_scope("copy_drain"):
        for c in range(7):
            cin(c, CCH).wait()
            cout(c, CCH).start()
            cout(c, CCH).wait()
            if c + 2 < 7:
                cin(c + 2, CCH).start()
            elif c + 2 == 7:
                @pl.when(is_last)
                def _():
                    cin(7, CTAIL_LAST).start()

                @pl.when(jnp.logical_not(is_last))
                def _():
                    cin(7, CTAIL).start()

        @pl.when(is_last)
        def _():
            cin(7, CTAIL_LAST).wait()
            cout(7, CTAIL_LAST).start()
            cout(7, CTAIL_LAST).wait()

        @pl.when(jnp.logical_not(is_last))
        def _():
            cin(7, CTAIL).wait()
            cout(7, CTAIL).start()
            cout(7, CTAIL).wait()

    with jax.named_scope("gs"):
        @pl.loop(0, MAXCH)
        def _(j):
            @pl.when(j < nch)
            def _():
                pltpu.sync_copy(val_hbm.at[win2_v.at[j]], vbuf_v)
                pltpu.sync_copy(vbuf_v, out_hbm.at[row2_v.at[j]])


@jax.jit
def _scatter_sc(mem, idx32, val):
    mesh = plsc.VectorSubcoreMesh(core_axis_name="c", subcore_axis_name="s")
    kfn = pl.kernel(
        _sc_body,
        out_type=jax.ShapeDtypeStruct((CAP, DIM), mem.dtype),
        mesh=mesh,
        compiler_params=pltpu.CompilerParams(
            use_tc_tiling_on_sc=False, needs_layout_passes=False),
        scratch_types=[
            pltpu.VMEM((BATCH,), jnp.int32),
            pltpu.VMEM((POS_PAD,), jnp.int32),
            pltpu.VMEM((LIST_PAD,), jnp.int32),
            pltpu.VMEM((LIST_PAD,), jnp.int32),
            pltpu.VMEM((MAXCH, CHUNK), jnp.int32),
            pltpu.VMEM((MAXCH, CHUNK), jnp.int32),
            pltpu.VMEM((CHUNK, DIM), jnp.float32),
            pltpu.VMEM((2, CCH, DIM), jnp.float32),
            pltpu.SemaphoreType.DMA,
            pltpu.SemaphoreType.DMA,
            pltpu.SemaphoreType.DMA,
            pltpu.SemaphoreType.DMA,
        ],
    )
    return kfn(mem, idx32, val)


def kernel(mem, idx, val):
    return _scatter_sc(mem, idx.astype(jnp.int32), val)
